# baseline (speedup 1.0000x reference)
import jax
import jax.numpy as jnp
from jax import lax
from jax.experimental import pallas as pl
from jax.experimental.pallas import tpu as pltpu

D = 1024
B = 4 * 4096
LAG = 128


def _tc_kernel(idx_ref, table_ref, out_ref, sem):
    def issue(i, _):
        r = idx_ref[i]
        pltpu.make_async_copy(table_ref.at[pl.ds(r, 1)], out_ref.at[pl.ds(i, 1)], sem).start()

        @pl.when(i >= LAG)
        def _():
            pltpu.make_async_copy(table_ref.at[pl.ds(0, 1)], out_ref.at[pl.ds(0, 1)], sem).wait()

        return 0

    lax.fori_loop(0, B, issue, 0)

    def drain(i, _):
        pltpu.make_async_copy(table_ref.at[pl.ds(0, 1)], out_ref.at[pl.ds(0, 1)], sem).wait()
        return 0

    lax.fori_loop(0, LAG, drain, 0)


@jax.jit
def _run(ids_flat, wte):
    return pl.pallas_call(
        _tc_kernel,
        out_shape=jax.ShapeDtypeStruct((B, D), jnp.float32),
        in_specs=[
            pl.BlockSpec(memory_space=pltpu.SMEM),
            pl.BlockSpec(memory_space=pl.ANY),
        ],
        out_specs=pl.BlockSpec(memory_space=pl.ANY),
        scratch_shapes=[pltpu.SemaphoreType.DMA],
    )(ids_flat, wte)


def kernel(input_ids, wte):
    ids_flat = input_ids.reshape(-1).astype(jnp.int32)
    out = _run(ids_flat, wte)
    return out.reshape(input_ids.shape + (D,))


# speedup vs baseline: 30.6287x; 30.6287x over previous
import functools

import jax
import jax.numpy as jnp
from jax import lax
from jax.experimental import pallas as pl
from jax.experimental.pallas import tpu as pltpu
from jax.experimental.pallas import tpu_sc as plsc

D = 1024
B = 4 * 4096
NW = 32
B_PER_W = B // NW
CHUNK = 16
N_CHUNKS = B_PER_W // CHUNK


NBUF = 6


def _emb_kernel(table_hbm, idx_hbm, out_hbm, idx_v, buf0, buf1, buf2, buf3, buf4, buf5,
                gsem0, gsem1, gsem2, gsem3, gsem4, gsem5,
                ssem0, ssem1, ssem2, ssem3, ssem4, ssem5):
    wid = lax.axis_index("s") * 2 + lax.axis_index("c")
    base = wid * B_PER_W
    pltpu.sync_copy(idx_hbm.at[wid], idx_v)

    bufs = (buf0, buf1, buf2, buf3, buf4, buf5)
    gsems = (gsem0, gsem1, gsem2, gsem3, gsem4, gsem5)
    ssems = (ssem0, ssem1, ssem2, ssem3, ssem4, ssem5)

    for j in range(NBUF - 1):
        pltpu.async_copy(table_hbm.at[idx_v.at[j]], bufs[j], gsems[j])
    A = NBUF - 1
    for i in range(N_CHUNKS):
        if i + A < N_CHUNKS:
            b = (i + A) % NBUF
            if i >= 1:
                pltpu.make_async_copy(bufs[b], out_hbm.at[pl.ds(0, CHUNK)], ssems[b]).wait()
            pltpu.async_copy(table_hbm.at[idx_v.at[i + A]], bufs[b], gsems[b])
        cur = i % NBUF
        pltpu.make_async_copy(table_hbm.at[idx_v.at[i]], bufs[cur], gsems[cur]).wait()
        pltpu.async_copy(bufs[cur], out_hbm.at[pl.ds((i * NW + wid) * CHUNK, CHUNK)], ssems[cur])
    for i in range(N_CHUNKS - NBUF, N_CHUNKS):
        b = i % NBUF
        pltpu.make_async_copy(bufs[b], out_hbm.at[pl.ds(0, CHUNK)], ssems[b]).wait()


@jax.jit
def _run(ids_grp, wte):
    mesh = plsc.VectorSubcoreMesh(core_axis_name="c", subcore_axis_name="s")
    k = functools.partial(
        pl.kernel,
        mesh=mesh,
        out_type=jax.ShapeDtypeStruct((B, D), jnp.float32),
        scratch_types=[
            pltpu.VMEM((N_CHUNKS, CHUNK), jnp.int32),
        ] + [pltpu.VMEM((CHUNK, D), jnp.float32)] * 6 + [pltpu.SemaphoreType.DMA] * 12,
    )(_emb_kernel)
    return k(wte, ids_grp)


def kernel(input_ids, wte):
    ids_grp = input_ids.reshape(N_CHUNKS, NW, CHUNK).transpose(1, 0, 2).astype(jnp.int32)
    out = _run(ids_grp, wte)
    return out.reshape(input_ids.shape + (D,))
